# baseline (speedup 1.0000x reference)
import functools

import jax
import jax.numpy as jnp
from jax import lax
from jax.experimental import pallas as pl
from jax.experimental.pallas import tpu as pltpu
from jax.experimental.pallas import tpu_sc as plsc

N = 10000
E = 320000
IN_DIM = 128
HID = 64
OUT3 = 32

NC = 2
NS = 16
NW = NC * NS

CH = 128
EPW = -(-E // NW)
KCH = 2 * (-(-EPW // (2 * CH)))
KTOT = KCH + 1
EP = NW * KCH * CH

NPAD = 10240
ROWS = NPAD // NS
BR = 512
GRID = NPAD // BR

_mesh = functools.partial(
    plsc.VectorSubcoreMesh, core_axis_name="c", subcore_axis_name="s")


@functools.partial(
    pl.kernel,
    out_type=jax.ShapeDtypeStruct((NW, NPAD), jnp.float32),
    mesh=_mesh(),
    compiler_params=pltpu.CompilerParams(needs_layout_passes=False),
    scratch_types=[
        pltpu.VMEM((KTOT, CH), jnp.int32),
        pltpu.VMEM((NPAD,), jnp.float32),
    ],
)
def _deg_hist(dstp_hbm, out_hbm, dstb, hist):
    cid = lax.axis_index("c")
    sid = lax.axis_index("s")
    wid = cid * NS + sid
    pltpu.sync_copy(dstp_hbm.at[wid], dstb)

    def zero_body(i, c):
        hist[pl.ds(i * 16, 16)] = jnp.zeros((16,), jnp.float32)
        return c

    lax.fori_loop(0, NPAD // 16, zero_body, 0)
    ones = jnp.ones((16,), jnp.float32)

    def chunk_body(j, c):
        def vec_body(k, c2):
            idx = dstb[j, pl.ds(k * 16, 16)]
            plsc.addupdate_scatter(hist, [idx], ones)
            return c2
        return lax.fori_loop(0, CH // 16, vec_body, c)

    lax.fori_loop(0, KCH, chunk_body, 0)
    pltpu.sync_copy(hist, out_hbm.at[wid])


def _make_agg(D):
    @functools.partial(
        pl.kernel,
        out_type=jax.ShapeDtypeStruct((NC, NPAD, D), jnp.float32),
        mesh=_mesh(),
        compiler_params=pltpu.CompilerParams(use_tc_tiling_on_sc=False),
        scratch_types=[
            pltpu.VMEM((KTOT, CH), jnp.int32),
            pltpu.VMEM((KTOT, CH), jnp.int32),
            pltpu.VMEM((CH, D), jnp.float32),
            pltpu.VMEM((CH, D), jnp.float32),
            pltpu.VMEM_SHARED((NPAD, D), jnp.float32),
            pltpu.VMEM_SHARED((NPAD, D), jnp.float32),
            pltpu.SemaphoreType.DMA,
            pltpu.SemaphoreType.DMA,
        ],
    )
    def agg(u_hbm, srcp_hbm, dstp_hbm, zeros_hbm, out_hbm,
            srcb, dstb, buf0, buf1, ustage, accum, sem0, sem1):
        cid = lax.axis_index("c")
        sid = lax.axis_index("s")
        wid = cid * NS + sid
        ca = pltpu.async_copy(srcp_hbm.at[wid], srcb, sem0)
        cb = pltpu.async_copy(dstp_hbm.at[wid], dstb, sem1)
        pltpu.sync_copy(u_hbm.at[pl.ds(sid * ROWS, ROWS)],
                        ustage.at[pl.ds(sid * ROWS, ROWS)])
        pltpu.sync_copy(zeros_hbm.at[pl.ds(sid * ROWS, ROWS)],
                        accum.at[pl.ds(sid * ROWS, ROWS)])
        ca.wait()
        cb.wait()
        plsc.subcore_barrier()

        pltpu.async_copy(ustage.at[srcb.at[0]], buf0, sem0)

        def body(i, c):
            j0 = 2 * i
            pltpu.async_copy(ustage.at[srcb.at[j0 + 1]], buf1, sem1)
            pltpu.make_async_copy(ustage.at[srcb.at[j0]], buf0, sem0).wait()
            pltpu.sync_copy(buf0, accum.at[dstb.at[j0]], add=True)
            pltpu.async_copy(ustage.at[srcb.at[j0 + 2]], buf0, sem0)
            pltpu.make_async_copy(ustage.at[srcb.at[j0 + 1]], buf1,
                                  sem1).wait()
            pltpu.sync_copy(buf1, accum.at[dstb.at[j0 + 1]], add=True)
            return c

        lax.fori_loop(0, KCH // 2, body, 0)
        pltpu.make_async_copy(ustage.at[srcb.at[KCH]], buf0, sem0).wait()
        plsc.subcore_barrier()
        pltpu.sync_copy(accum.at[pl.ds(sid * ROWS, ROWS)],
                        out_hbm.at[cid, pl.ds(sid * ROWS, ROWS)])

    return agg


_agg64 = _make_agg(HID)
_agg32 = _make_agg(OUT3)


_HI = lax.Precision.HIGHEST


def _dinv_of(degh_blk):
    deg = lax.dot_general(degh_blk, jnp.ones((NW, 1), jnp.float32),
                          (((0,), (0,)), ((), ())),
                          precision=_HI,
                          preferred_element_type=jnp.float32) + 1.0
    r = lax.rsqrt(deg)
    return r * (1.5 - 0.5 * deg * r * r)


def _k0_body(x_ref, w_ref, degT_ref, u_ref):
    dinv = _dinv_of(degT_ref[...])
    u_ref[...] = dinv * jnp.dot(x_ref[...], w_ref[...],
                                preferred_element_type=jnp.float32,
                                precision=_HI)


def _k0(xp, W1, degT):
    return pl.pallas_call(
        _k0_body,
        grid=(GRID,),
        in_specs=[
            pl.BlockSpec((BR, IN_DIM), lambda i: (i, 0)),
            pl.BlockSpec((IN_DIM, HID), lambda i: (0, 0)),
            pl.BlockSpec((NW, BR), lambda i: (0, i)),
        ],
        out_specs=pl.BlockSpec((BR, HID), lambda i: (i, 0)),
        out_shape=jax.ShapeDtypeStruct((NPAD, HID), jnp.float32),
    )(xp, W1, degT)


def _mid_body(s_ref, u_ref, degT_ref, b_ref, w_ref, o_ref):
    dinv = _dinv_of(degT_ref[...])
    h = jax.nn.relu(dinv * (s_ref[0] + s_ref[1] + u_ref[...]) + b_ref[...])
    o_ref[...] = dinv * jnp.dot(h, w_ref[...],
                                preferred_element_type=jnp.float32,
                                precision=_HI)


def _kmid(S, u, degT, b, W, dout):
    din = u.shape[1]
    return pl.pallas_call(
        _mid_body,
        grid=(GRID,),
        in_specs=[
            pl.BlockSpec((NC, BR, din), lambda i: (0, i, 0)),
            pl.BlockSpec((BR, din), lambda i: (i, 0)),
            pl.BlockSpec((NW, BR), lambda i: (0, i)),
            pl.BlockSpec((1, din), lambda i: (0, 0)),
            pl.BlockSpec((din, dout), lambda i: (0, 0)),
        ],
        out_specs=pl.BlockSpec((BR, dout), lambda i: (i, 0)),
        out_shape=jax.ShapeDtypeStruct((NPAD, dout), jnp.float32),
    )(S, u, degT, b, W)


def _k3_body(s_ref, u_ref, degT_ref, b_ref, wh1_ref, bh1_ref, wh2_ref,
             bh2_ref, y_ref, acc_ref):
    i = pl.program_id(0)
    dinv = _dinv_of(degT_ref[...])
    h = jax.nn.relu(dinv * (s_ref[0] + s_ref[1] + u_ref[...]) + b_ref[...])
    row = lax.broadcasted_iota(jnp.int32, (BR, 1), 0) + i * BR
    h = jnp.where(row < N, h, 0.0)
    part = jnp.sum(h.reshape(BR // 8, 8, OUT3), axis=0)

    @pl.when(i == 0)
    def _():
        acc_ref[...] = part

    @pl.when(i > 0)
    def _():
        acc_ref[...] = acc_ref[...] + part

    @pl.when(i == GRID - 1)
    def _():
        g = jnp.sum(acc_ref[...], axis=0, keepdims=True) * (1.0 / N)
        z = jax.nn.relu(jnp.dot(g, wh1_ref[...],
                                preferred_element_type=jnp.float32,
                                precision=_HI)
                        + bh1_ref[...])
        y_ref[...] = (jnp.dot(z, wh2_ref[...],
                              preferred_element_type=jnp.float32,
                              precision=_HI)
                      + bh2_ref[...])


def _k3(S, u, degT, b3, Wh1, bh1, Wh2, bh2):
    return pl.pallas_call(
        _k3_body,
        grid=(GRID,),
        in_specs=[
            pl.BlockSpec((NC, BR, OUT3), lambda i: (0, i, 0)),
            pl.BlockSpec((BR, OUT3), lambda i: (i, 0)),
            pl.BlockSpec((NW, BR), lambda i: (0, i)),
            pl.BlockSpec((1, OUT3), lambda i: (0, 0)),
            pl.BlockSpec((OUT3, OUT3), lambda i: (0, 0)),
            pl.BlockSpec((1, OUT3), lambda i: (0, 0)),
            pl.BlockSpec((OUT3, 1), lambda i: (0, 0)),
            pl.BlockSpec((1, 1), lambda i: (0, 0)),
        ],
        out_specs=pl.BlockSpec((1, 1), lambda i: (0, 0)),
        out_shape=jax.ShapeDtypeStruct((1, 1), jnp.float32),
        scratch_shapes=[pltpu.VMEM((8, OUT3), jnp.float32)],
    )(S, u, degT, b3, Wh1, bh1, Wh2, bh2)


def kernel(x, edge_index, batch, W1, b1, W2, b2, W3, b3, Wh1, bh1, Wh2, bh2):
    del batch
    xp = jnp.pad(x, ((0, NPAD - N), (0, 0)))
    pad = jnp.full((EP - E,), N, jnp.int32)
    dummy = jnp.full((NW, 1, CH), N, jnp.int32)
    srcp = jnp.concatenate(
        [jnp.concatenate([edge_index[0], pad]).reshape(NW, KCH, CH), dummy],
        axis=1)
    dstp = jnp.concatenate(
        [jnp.concatenate([edge_index[1], pad]).reshape(NW, KCH, CH), dummy],
        axis=1)
    z64 = jnp.zeros((NPAD, HID), jnp.float32)
    z32 = jnp.zeros((NPAD, OUT3), jnp.float32)

    degT = _deg_hist(dstp)
    u1 = _k0(xp, W1, degT)
    S1 = _agg64(u1, srcp, dstp, z64)
    u2 = _kmid(S1, u1, degT, b1.reshape(1, -1), W2, HID)
    S2 = _agg64(u2, srcp, dstp, z64)
    u3 = _kmid(S2, u2, degT, b2.reshape(1, -1), W3, OUT3)
    S3 = _agg32(u3, srcp, dstp, z32)
    y = _k3(S3, u3, degT, b3.reshape(1, -1), Wh1, bh1.reshape(1, -1),
            Wh2, bh2.reshape(1, -1))
    return y.reshape(1)


# speedup vs baseline: 1.0365x; 1.0365x over previous
import functools

import jax
import jax.numpy as jnp
from jax import lax
from jax.experimental import pallas as pl
from jax.experimental.pallas import tpu as pltpu
from jax.experimental.pallas import tpu_sc as plsc

N = 10000
E = 320000
IN_DIM = 128
HID = 64
OUT3 = 32

NC = 2
NS = 16
NW = NC * NS

CH = 128
EPW = -(-E // NW)
KCH = 2 * (-(-EPW // (2 * CH)))
KTOT = KCH + 1
EP = NW * KCH * CH

NPAD = 10240
ROWS = NPAD // NS
BR = 512
GRID = NPAD // BR

_mesh = functools.partial(
    plsc.VectorSubcoreMesh, core_axis_name="c", subcore_axis_name="s")


@functools.partial(
    pl.kernel,
    out_type=jax.ShapeDtypeStruct((NW, NPAD), jnp.float32),
    mesh=_mesh(),
    compiler_params=pltpu.CompilerParams(needs_layout_passes=False),
    scratch_types=[
        pltpu.VMEM((KTOT, CH), jnp.int32),
        pltpu.VMEM((NPAD,), jnp.float32),
    ],
)
def _deg_hist(dstp_hbm, out_hbm, dstb, hist):
    cid = lax.axis_index("c")
    sid = lax.axis_index("s")
    wid = cid * NS + sid
    pltpu.sync_copy(dstp_hbm.at[wid], dstb)

    def zero_body(i, c):
        hist[pl.ds(i * 16, 16)] = jnp.zeros((16,), jnp.float32)
        return c

    lax.fori_loop(0, NPAD // 16, zero_body, 0)
    ones = jnp.ones((16,), jnp.float32)

    def chunk_body(j, c):
        def vec_body(k, c2):
            idx = dstb[j, pl.ds(k * 16, 16)]
            plsc.addupdate_scatter(hist, [idx], ones)
            return c2
        return lax.fori_loop(0, CH // 16, vec_body, c)

    lax.fori_loop(0, KCH, chunk_body, 0)
    pltpu.sync_copy(hist, out_hbm.at[wid])


def _make_agg(D):
    @functools.partial(
        pl.kernel,
        out_type=jax.ShapeDtypeStruct((NC, NPAD, D), jnp.float32),
        mesh=_mesh(),
        compiler_params=pltpu.CompilerParams(use_tc_tiling_on_sc=False),
        scratch_types=[
            pltpu.VMEM((KTOT, CH), jnp.int32),
            pltpu.VMEM((KTOT, CH), jnp.int32),
            pltpu.VMEM((CH, D), jnp.float32),
            pltpu.VMEM((CH, D), jnp.float32),
            pltpu.VMEM_SHARED((NPAD, D), jnp.float32),
            pltpu.VMEM_SHARED((NPAD, D), jnp.float32),
            pltpu.SemaphoreType.DMA,
            pltpu.SemaphoreType.DMA,
        ],
    )
    def agg(u_hbm, srcp_hbm, dstp_hbm, zeros_hbm, out_hbm,
            srcb, dstb, buf0, buf1, ustage, accum, sem0, sem1):
        cid = lax.axis_index("c")
        sid = lax.axis_index("s")
        wid = cid * NS + sid
        ca = pltpu.async_copy(srcp_hbm.at[wid], srcb, sem0)
        cb = pltpu.async_copy(dstp_hbm.at[wid], dstb, sem1)
        pltpu.sync_copy(u_hbm.at[pl.ds(sid * ROWS, ROWS)],
                        ustage.at[pl.ds(sid * ROWS, ROWS)])
        pltpu.sync_copy(zeros_hbm.at[pl.ds(sid * ROWS, ROWS)],
                        accum.at[pl.ds(sid * ROWS, ROWS)])
        ca.wait()
        cb.wait()
        plsc.subcore_barrier()

        pltpu.async_copy(ustage.at[srcb.at[0]], buf0, sem0)

        def body(i, c):
            j0 = 2 * i
            pltpu.async_copy(ustage.at[srcb.at[j0 + 1]], buf1, sem1)
            pltpu.make_async_copy(ustage.at[srcb.at[j0]], buf0, sem0).wait()
            pltpu.sync_copy(buf0, accum.at[dstb.at[j0]], add=True)
            pltpu.async_copy(ustage.at[srcb.at[j0 + 2]], buf0, sem0)
            pltpu.make_async_copy(ustage.at[srcb.at[j0 + 1]], buf1,
                                  sem1).wait()
            pltpu.sync_copy(buf1, accum.at[dstb.at[j0 + 1]], add=True)
            return c

        lax.fori_loop(0, KCH // 2, body, 0)
        pltpu.make_async_copy(ustage.at[srcb.at[KCH]], buf0, sem0).wait()
        plsc.subcore_barrier()
        pltpu.sync_copy(accum.at[pl.ds(sid * ROWS, ROWS)],
                        out_hbm.at[cid, pl.ds(sid * ROWS, ROWS)])

    return agg


_agg64 = _make_agg(HID)
_agg32 = _make_agg(OUT3)


_HI = lax.Precision.HIGHEST


def _dinv_of(degh_blk):
    deg = lax.dot_general(degh_blk, jnp.ones((NW, 1), jnp.float32),
                          (((0,), (0,)), ((), ())),
                          precision=_HI,
                          preferred_element_type=jnp.float32) + 1.0
    r = lax.rsqrt(deg)
    return r * (1.5 - 0.5 * deg * r * r)


def _k0_body(x_ref, w_ref, degT_ref, u_ref):
    dinv = _dinv_of(degT_ref[...])
    u_ref[...] = dinv * jnp.dot(x_ref[...], w_ref[...],
                                preferred_element_type=jnp.float32)


def _k0(xp, W1, degT):
    return pl.pallas_call(
        _k0_body,
        grid=(GRID,),
        in_specs=[
            pl.BlockSpec((BR, IN_DIM), lambda i: (i, 0)),
            pl.BlockSpec((IN_DIM, HID), lambda i: (0, 0)),
            pl.BlockSpec((NW, BR), lambda i: (0, i)),
        ],
        out_specs=pl.BlockSpec((BR, HID), lambda i: (i, 0)),
        out_shape=jax.ShapeDtypeStruct((NPAD, HID), jnp.float32),
    )(xp, W1, degT)


def _mid_body(s_ref, u_ref, degT_ref, b_ref, w_ref, o_ref):
    dinv = _dinv_of(degT_ref[...])
    h = jax.nn.relu(dinv * (s_ref[0] + s_ref[1] + u_ref[...]) + b_ref[...])
    o_ref[...] = dinv * jnp.dot(h, w_ref[...],
                                preferred_element_type=jnp.float32)


def _kmid(S, u, degT, b, W, dout):
    din = u.shape[1]
    return pl.pallas_call(
        _mid_body,
        grid=(GRID,),
        in_specs=[
            pl.BlockSpec((NC, BR, din), lambda i: (0, i, 0)),
            pl.BlockSpec((BR, din), lambda i: (i, 0)),
            pl.BlockSpec((NW, BR), lambda i: (0, i)),
            pl.BlockSpec((1, din), lambda i: (0, 0)),
            pl.BlockSpec((din, dout), lambda i: (0, 0)),
        ],
        out_specs=pl.BlockSpec((BR, dout), lambda i: (i, 0)),
        out_shape=jax.ShapeDtypeStruct((NPAD, dout), jnp.float32),
    )(S, u, degT, b, W)


def _k3_body(s_ref, u_ref, degT_ref, b_ref, wh1_ref, bh1_ref, wh2_ref,
             bh2_ref, y_ref, acc_ref):
    i = pl.program_id(0)
    dinv = _dinv_of(degT_ref[...])
    h = jax.nn.relu(dinv * (s_ref[0] + s_ref[1] + u_ref[...]) + b_ref[...])
    row = lax.broadcasted_iota(jnp.int32, (BR, 1), 0) + i * BR
    h = jnp.where(row < N, h, 0.0)
    part = jnp.sum(h.reshape(BR // 8, 8, OUT3), axis=0)

    @pl.when(i == 0)
    def _():
        acc_ref[...] = part

    @pl.when(i > 0)
    def _():
        acc_ref[...] = acc_ref[...] + part

    @pl.when(i == GRID - 1)
    def _():
        g = jnp.sum(acc_ref[...], axis=0, keepdims=True) * (1.0 / N)
        z = jax.nn.relu(jnp.dot(g, wh1_ref[...],
                                preferred_element_type=jnp.float32)
                        + bh1_ref[...])
        y_ref[...] = (jnp.dot(z, wh2_ref[...],
                              preferred_element_type=jnp.float32)
                      + bh2_ref[...])


def _k3(S, u, degT, b3, Wh1, bh1, Wh2, bh2):
    return pl.pallas_call(
        _k3_body,
        grid=(GRID,),
        in_specs=[
            pl.BlockSpec((NC, BR, OUT3), lambda i: (0, i, 0)),
            pl.BlockSpec((BR, OUT3), lambda i: (i, 0)),
            pl.BlockSpec((NW, BR), lambda i: (0, i)),
            pl.BlockSpec((1, OUT3), lambda i: (0, 0)),
            pl.BlockSpec((OUT3, OUT3), lambda i: (0, 0)),
            pl.BlockSpec((1, OUT3), lambda i: (0, 0)),
            pl.BlockSpec((OUT3, 1), lambda i: (0, 0)),
            pl.BlockSpec((1, 1), lambda i: (0, 0)),
        ],
        out_specs=pl.BlockSpec((1, 1), lambda i: (0, 0)),
        out_shape=jax.ShapeDtypeStruct((1, 1), jnp.float32),
        scratch_shapes=[pltpu.VMEM((8, OUT3), jnp.float32)],
    )(S, u, degT, b3, Wh1, bh1, Wh2, bh2)


def kernel(x, edge_index, batch, W1, b1, W2, b2, W3, b3, Wh1, bh1, Wh2, bh2):
    del batch
    xp = jnp.pad(x, ((0, NPAD - N), (0, 0)))
    pad = jnp.full((EP - E,), N, jnp.int32)
    dummy = jnp.full((NW, 1, CH), N, jnp.int32)
    srcp = jnp.concatenate(
        [jnp.concatenate([edge_index[0], pad]).reshape(NW, KCH, CH), dummy],
        axis=1)
    dstp = jnp.concatenate(
        [jnp.concatenate([edge_index[1], pad]).reshape(NW, KCH, CH), dummy],
        axis=1)
    z64 = jnp.zeros((NPAD, HID), jnp.float32)
    z32 = jnp.zeros((NPAD, OUT3), jnp.float32)

    degT = _deg_hist(dstp)
    u1 = _k0(xp, W1, degT)
    S1 = _agg64(u1, srcp, dstp, z64)
    u2 = _kmid(S1, u1, degT, b1.reshape(1, -1), W2, HID)
    S2 = _agg64(u2, srcp, dstp, z64)
    u3 = _kmid(S2, u2, degT, b2.reshape(1, -1), W3, OUT3)
    S3 = _agg32(u3, srcp, dstp, z32)
    y = _k3(S3, u3, degT, b3.reshape(1, -1), Wh1, bh1.reshape(1, -1),
            Wh2, bh2.reshape(1, -1))
    return y.reshape(1)
